# baseline (speedup 1.0000x reference)
import jax
import jax.numpy as jnp
from jax.experimental import pallas as pl
from jax.experimental.pallas import tpu as pltpu

N_NODES = 10000
BLK = 1000


def _fused_mlp_kernel(x_ref, W1_ref, b1_ref, W2_ref, b2_ref, W3_ref, b3_ref,
                      Wo1_ref, bo1_ref, Wo2_ref, bo2_ref, out_ref, acc_ref):
    i = pl.program_id(0)

    @pl.when(i == 0)
    def _init():
        acc_ref[...] = jnp.zeros_like(acc_ref)

    h = jnp.dot(x_ref[...], W1_ref[...], preferred_element_type=jnp.float32)
    h = jnp.maximum(h + b1_ref[...], 0.0)
    h = jnp.dot(h, W2_ref[...], preferred_element_type=jnp.float32)
    h = jnp.maximum(h + b2_ref[...], 0.0)
    acc_ref[...] += jnp.sum(h, axis=0, keepdims=True)

    @pl.when(i == pl.num_programs(0) - 1)
    def _finish():
        g = acc_ref[...] * (1.0 / N_NODES)
        g = jnp.dot(g, W3_ref[...], preferred_element_type=jnp.float32) + b3_ref[...]
        p = jnp.dot(g, Wo1_ref[...], preferred_element_type=jnp.float32)
        p = jnp.maximum(p + bo1_ref[...], 0.0)
        out_ref[...] = (jnp.dot(p, Wo2_ref[...], preferred_element_type=jnp.float32)
                        + bo2_ref[...])


def kernel(x, edge_index, W1, b1, W2, b2, W3, b3, Wo1, bo1, Wo2, bo2):
    del edge_index
    b1 = b1.reshape(1, -1)
    b2 = b2.reshape(1, -1)
    b3 = b3.reshape(1, -1)
    bo1 = bo1.reshape(1, -1)
    bo2 = bo2.reshape(1, -1)

    grid = N_NODES // BLK
    full = lambda a: pl.BlockSpec(a.shape, lambda i: (0, 0))
    out = pl.pallas_call(
        _fused_mlp_kernel,
        grid=(grid,),
        in_specs=[
            pl.BlockSpec((BLK, x.shape[1]), lambda i: (i, 0)),
            full(W1), full(b1), full(W2), full(b2), full(W3), full(b3),
            full(Wo1), full(bo1), full(Wo2), full(bo2),
        ],
        out_specs=pl.BlockSpec((1, 1), lambda i: (0, 0)),
        out_shape=jax.ShapeDtypeStruct((1, 1), jnp.float32),
        scratch_shapes=[pltpu.VMEM((1, x.shape[1]), jnp.float32)],
    )(x, W1, b1, W2, b2, W3, b3, Wo1, bo1, Wo2, bo2)
    return out


# speedup vs baseline: 1.3142x; 1.3142x over previous
import jax
import jax.numpy as jnp
from jax.experimental import pallas as pl
from jax.experimental.pallas import tpu as pltpu

N_NODES = 10000
BLK = 2000


def _fused_mlp_kernel(x_ref, W1_ref, b1_ref, W2_ref, b2_ref, W3_ref, b3_ref,
                      Wo1_ref, bo1_ref, Wo2_ref, bo2_ref, out_ref, acc_ref):
    i = pl.program_id(0)

    @pl.when(i == 0)
    def _init():
        acc_ref[...] = jnp.zeros_like(acc_ref)

    h = jnp.dot(x_ref[...], W1_ref[...], preferred_element_type=jnp.float32)
    h = jnp.maximum(h + b1_ref[...], 0.0)
    h = jnp.dot(h, W2_ref[...], preferred_element_type=jnp.float32)
    h = jnp.maximum(h + b2_ref[...], 0.0)
    acc_ref[...] += jnp.sum(h, axis=0, keepdims=True)

    @pl.when(i == pl.num_programs(0) - 1)
    def _finish():
        g = acc_ref[...] * (1.0 / N_NODES)
        g = jnp.dot(g, W3_ref[...], preferred_element_type=jnp.float32) + b3_ref[...]
        p = jnp.dot(g, Wo1_ref[...], preferred_element_type=jnp.float32)
        p = jnp.maximum(p + bo1_ref[...], 0.0)
        out_ref[...] = (jnp.dot(p, Wo2_ref[...], preferred_element_type=jnp.float32)
                        + bo2_ref[...])


def kernel(x, edge_index, W1, b1, W2, b2, W3, b3, Wo1, bo1, Wo2, bo2):
    del edge_index
    b1 = b1.reshape(1, -1)
    b2 = b2.reshape(1, -1)
    b3 = b3.reshape(1, -1)
    bo1 = bo1.reshape(1, -1)
    bo2 = bo2.reshape(1, -1)

    grid = N_NODES // BLK
    full = lambda a: pl.BlockSpec(a.shape, lambda i: (0, 0))
    out = pl.pallas_call(
        _fused_mlp_kernel,
        grid=(grid,),
        in_specs=[
            pl.BlockSpec((BLK, x.shape[1]), lambda i: (i, 0)),
            full(W1), full(b1), full(W2), full(b2), full(W3), full(b3),
            full(Wo1), full(bo1), full(Wo2), full(bo2),
        ],
        out_specs=pl.BlockSpec((1, 1), lambda i: (0, 0)),
        out_shape=jax.ShapeDtypeStruct((1, 1), jnp.float32),
        scratch_shapes=[pltpu.VMEM((1, x.shape[1]), jnp.float32)],
    )(x, W1, b1, W2, b2, W3, b3, Wo1, bo1, Wo2, bo2)
    return out


# speedup vs baseline: 1.3685x; 1.0413x over previous
import jax
import jax.numpy as jnp
from jax.experimental import pallas as pl
from jax.experimental.pallas import tpu as pltpu

N_NODES = 10000
BLK = 5000


def _fused_mlp_kernel(x_ref, W1_ref, b1_ref, W2_ref, b2_ref, W3_ref, b3_ref,
                      Wo1_ref, bo1_ref, Wo2_ref, bo2_ref, out_ref, acc_ref):
    i = pl.program_id(0)

    @pl.when(i == 0)
    def _init():
        acc_ref[...] = jnp.zeros_like(acc_ref)

    h = jnp.dot(x_ref[...], W1_ref[...], preferred_element_type=jnp.float32)
    h = jnp.maximum(h + b1_ref[...], 0.0)
    h = jnp.dot(h, W2_ref[...], preferred_element_type=jnp.float32)
    h = jnp.maximum(h + b2_ref[...], 0.0)
    acc_ref[...] += jnp.sum(h, axis=0, keepdims=True)

    @pl.when(i == pl.num_programs(0) - 1)
    def _finish():
        g = acc_ref[...] * (1.0 / N_NODES)
        g = jnp.dot(g, W3_ref[...], preferred_element_type=jnp.float32) + b3_ref[...]
        p = jnp.dot(g, Wo1_ref[...], preferred_element_type=jnp.float32)
        p = jnp.maximum(p + bo1_ref[...], 0.0)
        out_ref[...] = (jnp.dot(p, Wo2_ref[...], preferred_element_type=jnp.float32)
                        + bo2_ref[...])


def kernel(x, edge_index, W1, b1, W2, b2, W3, b3, Wo1, bo1, Wo2, bo2):
    del edge_index
    b1 = b1.reshape(1, -1)
    b2 = b2.reshape(1, -1)
    b3 = b3.reshape(1, -1)
    bo1 = bo1.reshape(1, -1)
    bo2 = bo2.reshape(1, -1)

    grid = N_NODES // BLK
    full = lambda a: pl.BlockSpec(a.shape, lambda i: (0, 0))
    out = pl.pallas_call(
        _fused_mlp_kernel,
        grid=(grid,),
        in_specs=[
            pl.BlockSpec((BLK, x.shape[1]), lambda i: (i, 0)),
            full(W1), full(b1), full(W2), full(b2), full(W3), full(b3),
            full(Wo1), full(bo1), full(Wo2), full(bo2),
        ],
        out_specs=pl.BlockSpec((1, 1), lambda i: (0, 0)),
        out_shape=jax.ShapeDtypeStruct((1, 1), jnp.float32),
        scratch_shapes=[pltpu.VMEM((1, x.shape[1]), jnp.float32)],
    )(x, W1, b1, W2, b2, W3, b3, Wo1, bo1, Wo2, bo2)
    return out


# speedup vs baseline: 1.5383x; 1.1241x over previous
import jax
import jax.numpy as jnp
from jax.experimental import pallas as pl
from jax.experimental.pallas import tpu as pltpu

N_NODES = 10000
BLK = 10000
GRID = N_NODES // BLK


def _fused_mlp_kernel(x_ref, W1_ref, b1_ref, W2_ref, b2_ref, W3_ref, b3_ref,
                      Wo1_ref, bo1_ref, Wo2_ref, bo2_ref, out_ref, acc_ref):
    i = pl.program_id(0)

    @pl.when(i == 0)
    def _init():
        acc_ref[...] = jnp.zeros_like(acc_ref)

    h = jnp.dot(x_ref[...], W1_ref[...], preferred_element_type=jnp.float32)
    h = jnp.maximum(h + b1_ref[...], 0.0)
    h = jnp.dot(h, W2_ref[...], preferred_element_type=jnp.float32)
    h = jnp.maximum(h + b2_ref[...], 0.0)
    acc_ref[...] += jnp.sum(h, axis=0, keepdims=True)

    @pl.when(i == pl.num_programs(0) - 1)
    def _finish():
        g = acc_ref[...] * (1.0 / N_NODES)
        g = jnp.dot(g, W3_ref[...], preferred_element_type=jnp.float32) + b3_ref[...]
        p = jnp.dot(g, Wo1_ref[...], preferred_element_type=jnp.float32)
        p = jnp.maximum(p + bo1_ref[...], 0.0)
        out_ref[...] = (jnp.dot(p, Wo2_ref[...], preferred_element_type=jnp.float32)
                        + bo2_ref[...])


def kernel(x, edge_index, W1, b1, W2, b2, W3, b3, Wo1, bo1, Wo2, bo2):
    del edge_index
    b1 = b1.reshape(1, -1)
    b2 = b2.reshape(1, -1)
    b3 = b3.reshape(1, -1)
    bo1 = bo1.reshape(1, -1)
    bo2 = bo2.reshape(1, -1)

    full = lambda a: pl.BlockSpec(a.shape, lambda i: (0, 0))
    out = pl.pallas_call(
        _fused_mlp_kernel,
        grid=(GRID,),
        in_specs=[
            pl.BlockSpec((BLK, x.shape[1]), lambda i: (i, 0)),
            full(W1), full(b1), full(W2), full(b2), full(W3), full(b3),
            full(Wo1), full(bo1), full(Wo2), full(bo2),
        ],
        out_specs=pl.BlockSpec((1, 1), lambda i: (0, 0)),
        out_shape=jax.ShapeDtypeStruct((1, 1), jnp.float32),
        scratch_shapes=[pltpu.VMEM((1, x.shape[1]), jnp.float32)],
    )(x, W1, b1, W2, b2, W3, b3, Wo1, bo1, Wo2, bo2)
    return out
